# baseline (speedup 1.0000x reference)
import functools

import jax
import jax.numpy as jnp
from jax import lax
from jax.experimental import pallas as pl
from jax.experimental.pallas import tpu as pltpu
from jax.experimental.pallas import tpu_sc as plsc

N = 8192
D = 256
RB = 1024
NB = N // RB
K = 1024

NTILES = 16
CHUNK = N // NTILES
QUOTA = 96
NCAND = NTILES * QUOTA
BISECT_ITERS = 45


def _tri_body(x_ref, wcol_ref, wrow_ref, mc_ref, mr_ref):
    i = pl.program_id(0)
    j = pl.program_id(1)

    @pl.when(j >= i)
    def _():
        xa = x_ref[pl.ds(i * RB, RB), :]
        xb = x_ref[pl.ds(j * RB, RB), :]
        P = jax.lax.dot_general(
            xa, xb, (((1,), (1,)), ((), ())),
            preferred_element_type=jnp.float32)
        wc = wcol_ref[pl.ds(i * RB, RB), :]
        wr = wrow_ref[:, pl.ds(j * RB, RB)]
        G = P / (wc * wr)
        r = jax.lax.broadcasted_iota(jnp.int32, (RB, RB), 0)
        c = jax.lax.broadcasted_iota(jnp.int32, (RB, RB), 1)
        Gm = jnp.where((r == c) & (i == j), -jnp.inf, G)
        rowm = jnp.max(Gm, axis=1, keepdims=True)
        colm = jnp.max(Gm, axis=0, keepdims=True)
        oldc = mc_ref[pl.ds(i * RB, RB), :]
        mc_ref[pl.ds(i * RB, RB), :] = jnp.where(
            j == i, rowm, jnp.maximum(oldc, rowm))
        oldr = mr_ref[:, pl.ds(j * RB, RB)]
        mr_ref[:, pl.ds(j * RB, RB)] = jnp.where(
            i == 0, colm, jnp.maximum(oldr, colm))


def _rowmax_tri(x, w_col, w_row):
    return pl.pallas_call(
        _tri_body,
        grid=(NB, NB),
        in_specs=[
            pl.BlockSpec((N, D), lambda i, j: (0, 0)),
            pl.BlockSpec((N, 1), lambda i, j: (0, 0)),
            pl.BlockSpec((1, N), lambda i, j: (0, 0)),
        ],
        out_specs=[
            pl.BlockSpec((N, 1), lambda i, j: (0, 0)),
            pl.BlockSpec((1, N), lambda i, j: (0, 0)),
        ],
        out_shape=[
            jax.ShapeDtypeStruct((N, 1), jnp.float32),
            jax.ShapeDtypeStruct((1, N), jnp.float32),
        ],
    )(x, w_col, w_row)


def _bisect_body(mc_ref, mr_ref, m_ref, vk_ref):
    m = jnp.maximum(mc_ref[...], mr_ref[...])
    m_ref[...] = m

    def it(_, lohi):
        lo, hi = lohi
        mid = (lo + hi) * jnp.float32(0.5)
        cnt = jnp.sum((m > mid).astype(jnp.int32))
        pred = cnt < K
        return (jnp.where(pred, lo, mid), jnp.where(pred, mid, hi))

    lo, hi = lax.fori_loop(
        0, BISECT_ITERS, it, (jnp.float32(-2.0), jnp.float32(2.0)))
    vk_ref[0, 0] = hi


def _combine_bisect(mc2, mr2):
    return pl.pallas_call(
        _bisect_body,
        in_specs=[
            pl.BlockSpec((64, 128), lambda: (0, 0)),
            pl.BlockSpec((64, 128), lambda: (0, 0)),
        ],
        out_specs=[
            pl.BlockSpec((64, 128), lambda: (0, 0)),
            pl.BlockSpec(memory_space=pltpu.SMEM),
        ],
        out_shape=[
            jax.ShapeDtypeStruct((64, 128), jnp.float32),
            jax.ShapeDtypeStruct((1, 1), jnp.float32),
        ],
    )(mc2, mr2)


def _sc_select_impl(m_hbm, vk_hbm, candv_hbm, candi_hbm, mv, vkv, clv, cli):
    cid = lax.axis_index("c")
    sid = lax.axis_index("s")

    @pl.when(cid == 0)
    def _():
        t = sid
        pltpu.sync_copy(m_hbm.at[pl.ds(t * CHUNK, CHUNK)], mv)
        pltpu.sync_copy(vk_hbm, vkv)
        for q in range((QUOTA + 16) // 16):
            clv[pl.ds(q * 16, 16)] = jnp.full((16,), -jnp.inf, jnp.float32)
            cli[pl.ds(q * 16, 16)] = jnp.full((16,), N, jnp.int32)
        vk = vkv[...]
        base = t * CHUNK
        lane = lax.iota(jnp.int32, 16)
        off = jnp.int32(0)
        for k in range(CHUNK // 16):
            v = mv[pl.ds(k * 16, 16)]
            msk = v >= vk
            key = lane + jnp.where(msk, jnp.int32(0), jnp.int32(1024))
            _, vs = plsc.sort_key_val(key, v)
            _, gs = plsc.sort_key_val(key, lane + (base + k * 16))
            offc = jnp.minimum(off, QUOTA)
            clv[pl.ds(offc, 16)] = vs
            cli[pl.ds(offc, 16)] = gs
            pc = plsc.cumsum(jnp.where(msk, jnp.int32(1), jnp.int32(0)))
            off = off + pc[15]
        pltpu.sync_copy(clv.at[pl.ds(0, QUOTA)],
                        candv_hbm.at[pl.ds(t * QUOTA, QUOTA)])
        pltpu.sync_copy(cli.at[pl.ds(0, QUOTA)],
                        candi_hbm.at[pl.ds(t * QUOTA, QUOTA)])


@functools.cache
def _sc_select_kernel():
    return pl.kernel(
        _sc_select_impl,
        out_type=[
            jax.ShapeDtypeStruct((NCAND,), jnp.float32),
            jax.ShapeDtypeStruct((NCAND,), jnp.int32),
        ],
        mesh=plsc.VectorSubcoreMesh(
            core_axis_name="c", subcore_axis_name="s"),
        compiler_params=pltpu.CompilerParams(needs_layout_passes=False),
        scratch_types=[
            pltpu.VMEM((CHUNK,), jnp.float32),
            pltpu.VMEM((16,), jnp.float32),
            pltpu.VMEM((QUOTA + 16,), jnp.float32),
            pltpu.VMEM((QUOTA + 16,), jnp.int32),
        ],
    )


def _sc_select(m, vk16):
    return _sc_select_kernel()(m, vk16)


def _rank_place_body(cvr_ref, cir_ref, cvc_ref, cic_ref,
                     vals_ref, inds_ref, rank_ref):
    rb = pl.program_id(0)

    @pl.when(rb == 0)
    def _():
        cvr = cvr_ref[...]
        cir = cir_ref[...]
        cvc = cvc_ref[...]
        cic = cic_ref[...]
        beats = (cvr > cvc) | ((cvr == cvc) & (cir < cic))
        rank_ref[...] = jnp.sum(beats.astype(jnp.int32), axis=1,
                                keepdims=True)

    rank = rank_ref[...]
    r_row = jax.lax.broadcasted_iota(jnp.int32, (1, 128), 1) + rb * 128
    onehot = (rank == r_row).astype(jnp.float32)
    cvc = cvc_ref[...]
    cic = cic_ref[...]
    vals_ref[...] = jnp.max(
        jnp.where(rank == r_row, cvc, -jnp.inf), axis=0, keepdims=True)
    inds_ref[...] = jnp.sum(
        onehot * cic.astype(jnp.float32), axis=0, keepdims=True
    ).astype(jnp.int32)


def _rank_place(cv_row, ci_row, cv_col, ci_col):
    return pl.pallas_call(
        _rank_place_body,
        grid=(K // 128,),
        in_specs=[
            pl.BlockSpec((1, NCAND), lambda rb: (0, 0)),
            pl.BlockSpec((1, NCAND), lambda rb: (0, 0)),
            pl.BlockSpec((NCAND, 1), lambda rb: (0, 0)),
            pl.BlockSpec((NCAND, 1), lambda rb: (0, 0)),
        ],
        out_specs=[
            pl.BlockSpec((1, 128), lambda rb: (0, rb)),
            pl.BlockSpec((1, 128), lambda rb: (0, rb)),
        ],
        out_shape=[
            jax.ShapeDtypeStruct((1, K), jnp.float32),
            jax.ShapeDtypeStruct((1, K), jnp.int32),
        ],
        scratch_shapes=[pltpu.VMEM((NCAND, 1), jnp.int32)],
    )(cv_row, ci_row, cv_col, ci_col)


def kernel(x, nb_selected):
    w = jnp.sqrt(jnp.sum(x * x, axis=1, keepdims=True))
    mc, mr = _rowmax_tri(x, w, w.reshape(1, N))
    m2d, vk = _combine_bisect(mc.reshape(64, 128), mr.reshape(64, 128))
    vk16 = jnp.broadcast_to(vk.reshape(()), (16,))
    candv, candi = _sc_select(m2d.reshape(N), vk16)
    vals, inds = _rank_place(candv.reshape(1, NCAND), candi.reshape(1, NCAND),
                             candv.reshape(NCAND, 1), candi.reshape(NCAND, 1))
    return vals.reshape(K), inds.reshape(K)
